# baseline (speedup 1.0000x reference)
import functools

import jax
import jax.numpy as jnp
from jax import lax
from jax.experimental import pallas as pl
from jax.experimental.pallas import tpu as pltpu
from jax.experimental.pallas import tpu_sc as plsc

_NC = 2
_NS = 16
_NW = _NC * _NS
_LANES = 16

_CHUNK_ROWS = 16
_PAIR = 2
_NGRP = 3
_PREF = 2
_UNROLL = 4


def _make_sc_add(B, S, D):
    R = S // _NW
    C = _CHUNK_ROWS
    NCH = R // C
    NP = B // _PAIR
    NSTEP = NCH * NP
    SL = D // _LANES
    mesh = plsc.VectorSubcoreMesh(core_axis_name="c", subcore_axis_name="s")

    @functools.partial(
        pl.kernel,
        mesh=mesh,
        out_type=jax.ShapeDtypeStruct((B, S, D), jnp.float32),
        compiler_params=pltpu.CompilerParams(use_tc_tiling_on_sc=True),
        scratch_types=[
            pltpu.VMEM((_NGRP, _PAIR, C, D), jnp.float32),
            pltpu.VMEM((2, C, D), jnp.float32),
            pltpu.SemaphoreType.DMA,
            pltpu.SemaphoreType.DMA,
            pltpu.SemaphoreType.DMA,
        ],
    )
    def sc_add(tok_hbm, tab_hbm, out_hbm, tok_v, tab_v, in_sem, out_sem, tab_sem):
        wid = lax.axis_index("s") * _NC + lax.axis_index("c")
        row0 = wid * R

        def start_in(step):
            c, p = divmod(step, NP)
            g = step % _NGRP
            rows = row0 + c * C
            waits = []
            if p == 0:
                waits.append(
                    pltpu.async_copy(
                        tab_hbm.at[pl.ds(rows, C)], tab_v.at[c % 2], tab_sem
                    )
                )
            for i in range(_PAIR):
                waits.append(
                    pltpu.async_copy(
                        tok_hbm.at[p * _PAIR + i, pl.ds(rows, C)],
                        tok_v.at[g, i],
                        in_sem,
                    )
                )
            return waits

        in_waits = []
        out_waits = []
        for k in range(min(_PREF, NSTEP)):
            in_waits.append(start_in(k))

        for step in range(NSTEP):
            c, p = divmod(step, NP)
            g = step % _NGRP
            rows = row0 + c * C
            for w in in_waits.pop(0):
                w.wait()

            @plsc.parallel_loop(0, C * SL, unroll=_UNROLL)
            def add_body(i, g=g, tb=c % 2):
                r = i // SL
                col = (i - r * SL) * _LANES
                sl = pl.ds(col, _LANES)
                t = tab_v[tb, r, sl]
                for k in range(_PAIR):
                    tok_v[g, k, r, sl] = tok_v[g, k, r, sl] + t

            out_waits.append(
                [
                    pltpu.async_copy(
                        tok_v.at[g, i],
                        out_hbm.at[p * _PAIR + i, pl.ds(rows, C)],
                        out_sem,
                    )
                    for i in range(_PAIR)
                ]
            )
            nxt = step + _PREF
            if nxt < NSTEP:
                if len(out_waits) > _NGRP - _PREF:
                    for w in out_waits.pop(0):
                        w.wait()
                in_waits.append(start_in(nxt))
        for ws in out_waits:
            for w in ws:
                w.wait()

    return sc_add


def kernel(encoded_tokens, pos_table):
    B, S, D = encoded_tokens.shape
    sc_add = _make_sc_add(B, S, D)
    return sc_add(encoded_tokens, pos_table)


# speedup vs baseline: 1.0533x; 1.0533x over previous
import functools

import jax
import jax.numpy as jnp
from jax import lax
from jax.experimental import pallas as pl
from jax.experimental.pallas import tpu as pltpu
from jax.experimental.pallas import tpu_sc as plsc

_NC = 2
_NS = 16
_NW = _NC * _NS
_LANES = 16

_CHUNK_ROWS = 8
_NGRP = 4
_PREF = 3
_UNROLL = 8


def _make_sc_add(B, S, D):
    R = S // _NW
    C = _CHUNK_ROWS
    NCH = R // C
    SL = D // _LANES
    mesh = plsc.VectorSubcoreMesh(core_axis_name="c", subcore_axis_name="s")

    @functools.partial(
        pl.kernel,
        mesh=mesh,
        out_type=jax.ShapeDtypeStruct((B, S, D), jnp.float32),
        compiler_params=pltpu.CompilerParams(
            use_tc_tiling_on_sc=True,
            skip_device_barrier=True,
        ),
        scratch_types=[
            pltpu.VMEM((_NGRP, B, C, D), jnp.float32),
            pltpu.VMEM((_NGRP, C, D), jnp.float32),
            pltpu.SemaphoreType.DMA,
            pltpu.SemaphoreType.DMA,
            pltpu.SemaphoreType.DMA,
        ],
    )
    def sc_add(tok_hbm, tab_hbm, out_hbm, tok_v, tab_v, in_sem, out_sem, tab_sem):
        wid = lax.axis_index("s") * _NC + lax.axis_index("c")
        row0 = wid * R

        def start_in(c):
            g = c % _NGRP
            rows = row0 + c * C
            waits = [
                pltpu.async_copy(
                    tab_hbm.at[pl.ds(rows, C)], tab_v.at[g], tab_sem
                )
            ]
            for b in range(B):
                waits.append(
                    pltpu.async_copy(
                        tok_hbm.at[b, pl.ds(rows, C)], tok_v.at[g, b], in_sem
                    )
                )
            return waits

        in_waits = []
        out_waits = []
        for c in range(min(_PREF, NCH)):
            in_waits.append(start_in(c))

        for c in range(NCH):
            g = c % _NGRP
            rows = row0 + c * C
            for w in in_waits.pop(0):
                w.wait()

            @plsc.parallel_loop(0, C * SL, unroll=_UNROLL)
            def add_body(i, g=g):
                r = i // SL
                col = (i - r * SL) * _LANES
                sl = pl.ds(col, _LANES)
                t = tab_v[g, r, sl]
                for b in range(B):
                    tok_v[g, b, r, sl] = tok_v[g, b, r, sl] + t

            out_waits.append(
                [
                    pltpu.async_copy(
                        tok_v.at[g, b], out_hbm.at[b, pl.ds(rows, C)], out_sem
                    )
                    for b in range(B)
                ]
            )
            nxt = c + _PREF
            if nxt < NCH:
                if len(out_waits) > _NGRP - _PREF:
                    for w in out_waits.pop(0):
                        w.wait()
                in_waits.append(start_in(nxt))
        for ws in out_waits:
            for w in ws:
                w.wait()

    return sc_add


def kernel(encoded_tokens, pos_table):
    B, S, D = encoded_tokens.shape
    sc_add = _make_sc_add(B, S, D)
    return sc_add(encoded_tokens, pos_table)


# speedup vs baseline: 1.0561x; 1.0027x over previous
import functools

import jax
import jax.numpy as jnp
from jax import lax
from jax.experimental import pallas as pl
from jax.experimental.pallas import tpu as pltpu
from jax.experimental.pallas import tpu_sc as plsc

_NC = 2
_NS = 16
_NW = _NC * _NS
_LANES = 16

_CHUNK_ROWS = 8
_NGRP = 4
_PREF = 3
_UNROLL = 8


def _make_sc_add(B, S, D):
    R = S // _NW
    C = _CHUNK_ROWS
    NCH = R // C
    SL = D // _LANES
    mesh = plsc.VectorSubcoreMesh(core_axis_name="c", subcore_axis_name="s")

    @functools.partial(
        pl.kernel,
        mesh=mesh,
        out_type=jax.ShapeDtypeStruct((B, S, D), jnp.float32),
        compiler_params=pltpu.CompilerParams(
            use_tc_tiling_on_sc=True,
            skip_device_barrier=True,
            disable_bounds_checks=True,
            disable_semaphore_checks=True,
        ),
        scratch_types=[
            pltpu.VMEM((_NGRP, B, C, D), jnp.float32),
            pltpu.VMEM((_NGRP, C, D), jnp.float32),
            pltpu.SemaphoreType.DMA,
            pltpu.SemaphoreType.DMA,
            pltpu.SemaphoreType.DMA,
        ],
    )
    def sc_add(tok_hbm, tab_hbm, out_hbm, tok_v, tab_v, in_sem, out_sem, tab_sem):
        wid = lax.axis_index("s") * _NC + lax.axis_index("c")
        row0 = wid * R

        def start_in(c):
            g = c % _NGRP
            rows = row0 + c * C
            waits = [
                pltpu.async_copy(
                    tab_hbm.at[pl.ds(rows, C)], tab_v.at[g], tab_sem
                )
            ]
            for b in range(B):
                waits.append(
                    pltpu.async_copy(
                        tok_hbm.at[b, pl.ds(rows, C)], tok_v.at[g, b], in_sem
                    )
                )
            return waits

        in_waits = []
        out_waits = []
        for c in range(min(_PREF, NCH)):
            in_waits.append(start_in(c))

        for c in range(NCH):
            g = c % _NGRP
            rows = row0 + c * C
            for w in in_waits.pop(0):
                w.wait()

            @plsc.parallel_loop(0, C * SL, unroll=_UNROLL)
            def add_body(i, g=g):
                r = i // SL
                col = (i - r * SL) * _LANES
                sl = pl.ds(col, _LANES)
                t = tab_v[g, r, sl]
                for b in range(B):
                    tok_v[g, b, r, sl] = tok_v[g, b, r, sl] + t

            out_waits.append(
                [
                    pltpu.async_copy(
                        tok_v.at[g, b], out_hbm.at[b, pl.ds(rows, C)], out_sem
                    )
                    for b in range(B)
                ]
            )
            nxt = c + _PREF
            if nxt < NCH:
                if len(out_waits) > _NGRP - _PREF:
                    for w in out_waits.pop(0):
                        w.wait()
                in_waits.append(start_in(nxt))
        for ws in out_waits:
            for w in ws:
                w.wait()

    return sc_add


def kernel(encoded_tokens, pos_table):
    B, S, D = encoded_tokens.shape
    sc_add = _make_sc_add(B, S, D)
    return sc_add(encoded_tokens, pos_table)
